# initial kernel scaffold (unmeasured)
import jax
import jax.numpy as jnp
from jax import lax
from jax.experimental import pallas as pl
from jax.experimental.pallas import tpu as pltpu

N_DEV = 4


def _ring_all_gather(x_shard):
    m_per, k = x_shard.shape

    def body(x_ref, xg_ref, copy_sem, send_sems, recv_sems):
        my = lax.axis_index("i")
        left = lax.rem(my + N_DEV - 1, N_DEV)
        right = lax.rem(my + 1, N_DEV)

        barrier_sem = pltpu.get_barrier_semaphore()
        for nbr in (left, right):
            pl.semaphore_signal(
                barrier_sem, inc=1,
                device_id=(nbr,), device_id_type=pl.DeviceIdType.MESH,
            )
        pl.semaphore_wait(barrier_sem, 2)

        local = pltpu.make_async_copy(
            x_ref, xg_ref.at[pl.ds(my * m_per, m_per), :], copy_sem
        )
        local.start()
        local.wait()

        for h in range(N_DEV - 1):
            origin = lax.rem(my + N_DEV - h, N_DEV) if h else my
            sl = pl.ds(origin * m_per, m_per)
            rdma = pltpu.make_async_remote_copy(
                src_ref=xg_ref.at[sl, :],
                dst_ref=xg_ref.at[sl, :],
                send_sem=send_sems.at[h],
                recv_sem=recv_sems.at[h],
                device_id=(right,),
                device_id_type=pl.DeviceIdType.MESH,
            )
            rdma.start()
            rdma.wait()

    return pl.pallas_call(
        body,
        out_shape=jax.ShapeDtypeStruct((N_DEV * m_per, k), x_shard.dtype),
        in_specs=[pl.BlockSpec(memory_space=pltpu.ANY)],
        out_specs=pl.BlockSpec(memory_space=pltpu.ANY),
        scratch_shapes=[
            pltpu.SemaphoreType.DMA,
            pltpu.SemaphoreType.DMA((N_DEV - 1,)),
            pltpu.SemaphoreType.DMA((N_DEV - 1,)),
        ],
        compiler_params=pltpu.CompilerParams(collective_id=0),
    )(x_shard)


def _gemm_silu(x_full, w):
    m, k = x_full.shape
    n = w.shape[1]
    tm = 256

    def body(x_ref, w_ref, o_ref):
        y = jnp.dot(x_ref[...], w_ref[...], preferred_element_type=jnp.float32)
        o_ref[...] = y * jax.nn.sigmoid(y)

    return pl.pallas_call(
        body,
        grid=(m // tm,),
        in_specs=[
            pl.BlockSpec((tm, k), lambda i: (i, 0)),
            pl.BlockSpec((k, n), lambda i: (0, 0)),
        ],
        out_specs=pl.BlockSpec((tm, n), lambda i: (i, 0)),
        out_shape=jax.ShapeDtypeStruct((m, n), jnp.float32),
    )(x_full, w)


def kernel(x, w_mat):
    x_full = _ring_all_gather(x)
    return _gemm_silu(x_full, w_mat)


# baseline (device time: 4374595 ns/iter reference)
import jax
import jax.numpy as jnp
from jax import lax
from jax.experimental import pallas as pl
from jax.experimental.pallas import tpu as pltpu

N_DEV = 4


def _ring_all_gather(x_shard):
    m_per, k = x_shard.shape

    def body(x_ref, xg_ref, copy_sem, send_sems, recv_sems):
        my = lax.axis_index("i")
        left = lax.rem(my + N_DEV - 1, N_DEV)
        right = lax.rem(my + 1, N_DEV)

        barrier_sem = pltpu.get_barrier_semaphore()
        for nbr in (left, right):
            pl.semaphore_signal(
                barrier_sem, inc=1,
                device_id=(nbr,), device_id_type=pl.DeviceIdType.MESH,
            )
        pl.semaphore_wait(barrier_sem, 2)

        local = pltpu.make_async_copy(
            x_ref, xg_ref.at[pl.ds(my * m_per, m_per), :], copy_sem
        )
        local.start()
        local.wait()

        for h in range(N_DEV - 1):
            origin = lax.rem(my + N_DEV - h, N_DEV) if h else my
            sl = pl.ds(origin * m_per, m_per)
            rdma = pltpu.make_async_remote_copy(
                src_ref=xg_ref.at[sl, :],
                dst_ref=xg_ref.at[sl, :],
                send_sem=send_sems.at[h],
                recv_sem=recv_sems.at[h],
                device_id=(right,),
                device_id_type=pl.DeviceIdType.MESH,
            )
            rdma.start()
            rdma.wait()

    return pl.pallas_call(
        body,
        out_shape=jax.ShapeDtypeStruct((N_DEV * m_per, k), x_shard.dtype),
        in_specs=[pl.BlockSpec(memory_space=pl.ANY)],
        out_specs=pl.BlockSpec(memory_space=pl.ANY),
        scratch_shapes=[
            pltpu.SemaphoreType.DMA,
            pltpu.SemaphoreType.DMA((N_DEV - 1,)),
            pltpu.SemaphoreType.DMA((N_DEV - 1,)),
        ],
        compiler_params=pltpu.CompilerParams(collective_id=0),
    )(x_shard)


def _gemm_silu(x_full, w):
    m, k = x_full.shape
    n = w.shape[1]
    tm = 256

    def body(x_ref, w_ref, o_ref):
        y = jnp.dot(x_ref[...], w_ref[...], preferred_element_type=jnp.float32)
        o_ref[...] = y * jax.nn.sigmoid(y)

    return pl.pallas_call(
        body,
        grid=(m // tm,),
        in_specs=[
            pl.BlockSpec((tm, k), lambda i: (i, 0)),
            pl.BlockSpec((k, n), lambda i: (0, 0)),
        ],
        out_specs=pl.BlockSpec((tm, n), lambda i: (i, 0)),
        out_shape=jax.ShapeDtypeStruct((m, n), jnp.float32),
        compiler_params=pltpu.CompilerParams(
            vmem_limit_bytes=100 * 1024 * 1024,
        ),
    )(x_full, w)


def kernel(x, w_mat):
    x_full = _ring_all_gather(x)
    return _gemm_silu(x_full, w_mat)


# device time: 2211833 ns/iter; 1.9778x vs baseline; 1.9778x over previous
import jax
import jax.numpy as jnp
from jax import lax
from jax.experimental import pallas as pl
from jax.experimental.pallas import tpu as pltpu

N_DEV = 4


def _ring_all_gather(x_shard):
    m_per, k = x_shard.shape

    m_half = m_per // 2

    def body(x_ref, xg_ref, copy_sem, send_r, recv_r, send_l, recv_l):
        my = lax.axis_index("i")
        left = lax.rem(my + N_DEV - 1, N_DEV)
        right = lax.rem(my + 1, N_DEV)

        barrier_sem = pltpu.get_barrier_semaphore()
        for nbr in (left, right):
            pl.semaphore_signal(
                barrier_sem, inc=1,
                device_id=(nbr,), device_id_type=pl.DeviceIdType.MESH,
            )
        pl.semaphore_wait(barrier_sem, 2)

        local = pltpu.make_async_copy(
            x_ref, xg_ref.at[pl.ds(my * m_per, m_per), :], copy_sem
        )
        local.start()

        for h in range(N_DEV - 1):
            o_r = lax.rem(my + N_DEV - h, N_DEV)
            o_l = lax.rem(my + h, N_DEV)
            sl_r = pl.ds(o_r * m_per, m_half)
            sl_l = pl.ds(o_l * m_per + m_half, m_half)
            src_r = x_ref.at[pl.ds(0, m_half), :] if h == 0 else xg_ref.at[sl_r, :]
            src_l = x_ref.at[pl.ds(m_half, m_half), :] if h == 0 else xg_ref.at[sl_l, :]
            rdma_r = pltpu.make_async_remote_copy(
                src_ref=src_r,
                dst_ref=xg_ref.at[sl_r, :],
                send_sem=send_r.at[h],
                recv_sem=recv_r.at[h],
                device_id=(right,),
                device_id_type=pl.DeviceIdType.MESH,
            )
            rdma_l = pltpu.make_async_remote_copy(
                src_ref=src_l,
                dst_ref=xg_ref.at[sl_l, :],
                send_sem=send_l.at[h],
                recv_sem=recv_l.at[h],
                device_id=(left,),
                device_id_type=pl.DeviceIdType.MESH,
            )
            rdma_r.start()
            rdma_l.start()
            rdma_r.wait()
            rdma_l.wait()

        local.wait()

    return pl.pallas_call(
        body,
        out_shape=jax.ShapeDtypeStruct((N_DEV * m_per, k), x_shard.dtype),
        in_specs=[pl.BlockSpec(memory_space=pl.ANY)],
        out_specs=pl.BlockSpec(memory_space=pl.ANY),
        scratch_shapes=[
            pltpu.SemaphoreType.DMA,
            pltpu.SemaphoreType.DMA((N_DEV - 1,)),
            pltpu.SemaphoreType.DMA((N_DEV - 1,)),
            pltpu.SemaphoreType.DMA((N_DEV - 1,)),
            pltpu.SemaphoreType.DMA((N_DEV - 1,)),
        ],
        compiler_params=pltpu.CompilerParams(collective_id=0),
    )(x_shard)


def _gemm_silu(x_full, w):
    m, k = x_full.shape
    n = w.shape[1]
    tm = 256

    def body(x_ref, w_ref, o_ref):
        y = jnp.dot(x_ref[...], w_ref[...], preferred_element_type=jnp.float32)
        o_ref[...] = y * jax.nn.sigmoid(y)

    return pl.pallas_call(
        body,
        grid=(m // tm,),
        in_specs=[
            pl.BlockSpec((tm, k), lambda i: (i, 0)),
            pl.BlockSpec((k, n), lambda i: (0, 0)),
        ],
        out_specs=pl.BlockSpec((tm, n), lambda i: (i, 0)),
        out_shape=jax.ShapeDtypeStruct((m, n), jnp.float32),
        compiler_params=pltpu.CompilerParams(
            vmem_limit_bytes=100 * 1024 * 1024,
        ),
    )(x_full, w)


def kernel(x, w_mat):
    x_full = _ring_all_gather(x)
    return _gemm_silu(x_full, w_mat)


# device time: 1821465 ns/iter; 2.4017x vs baseline; 1.2143x over previous
import jax
import jax.numpy as jnp
from jax import lax
from jax.experimental import pallas as pl
from jax.experimental.pallas import tpu as pltpu

N_DEV = 4
TM = 128


def kernel(x, w_mat):
    m_x, k = x.shape
    _, n_w = w_mat.shape
    nh = n_w // 2
    n_xt = m_x // TM

    def body(x_ref, w_ref, out_ref, wg_ref, wv_ref, xv_ref, yv_ref,
             split_sems, wv_sems, xv_sems, out_sems,
             send_r, recv_r, send_l, recv_l, a2a_send, a2a_recv):
        my = lax.axis_index("i")
        left = lax.rem(my + N_DEV - 1, N_DEV)
        right = lax.rem(my + 1, N_DEV)
        opp = lax.rem(my + 2, N_DEV)

        barrier_sem = pltpu.get_barrier_semaphore()
        for nbr in (left, right, opp):
            pl.semaphore_signal(
                barrier_sem, inc=1,
                device_id=(nbr,), device_id_type=pl.DeviceIdType.MESH,
            )
        pl.semaphore_wait(barrier_sem, 3)

        splits = []
        for half in range(2):
            c = pltpu.make_async_copy(
                w_ref.at[:, pl.ds(half * nh, nh)],
                wg_ref.at[my, half],
                split_sems.at[half],
            )
            c.start()
            splits.append(c)
        for c in splits:
            c.wait()

        def start_hop(h):
            o_r = lax.rem(my + N_DEV - h, N_DEV)
            o_l = lax.rem(my + h, N_DEV)
            rdma_r = pltpu.make_async_remote_copy(
                src_ref=wg_ref.at[o_r, 0],
                dst_ref=wg_ref.at[o_r, 0],
                send_sem=send_r.at[h],
                recv_sem=recv_r.at[h],
                device_id=(right,),
                device_id_type=pl.DeviceIdType.MESH,
            )
            rdma_l = pltpu.make_async_remote_copy(
                src_ref=wg_ref.at[o_l, 1],
                dst_ref=wg_ref.at[o_l, 1],
                send_sem=send_l.at[h],
                recv_sem=recv_l.at[h],
                device_id=(left,),
                device_id_type=pl.DeviceIdType.MESH,
            )
            rdma_r.start()
            rdma_l.start()
            return rdma_r, rdma_l

        def a2a_descriptor(slot, target, src_rows, half):
            return pltpu.make_async_remote_copy(
                src_ref=yv_ref.at[half],
                dst_ref=out_ref.at[pl.ds(src_rows * m_x, m_x),
                                   pl.ds(half * nh, nh)],
                send_sem=a2a_send.at[slot],
                recv_sem=a2a_recv.at[slot],
                device_id=(target,),
                device_id_type=pl.DeviceIdType.MESH,
            )

        def window(i_a, i_b):
            cw = []
            for half, i_dev in ((0, i_a), (1, i_b)):
                c = pltpu.make_async_copy(
                    wg_ref.at[i_dev, half], wv_ref.at[half], wv_sems.at[half]
                )
                c.start()
                cw.append(c)
            def x_tile_copy(t, slot):
                return pltpu.make_async_copy(
                    x_ref.at[pl.ds(t * TM, TM), :],
                    xv_ref.at[slot],
                    xv_sems.at[slot],
                )

            x_tile_copy(0, 0).start()
            for c in cw:
                c.wait()

            def tile_body(t, _):
                slot = lax.rem(t, 2)
                nxt_slot = lax.rem(t + 1, 2)

                @pl.when(t + 1 < n_xt)
                def _():
                    x_tile_copy(t + 1, nxt_slot).start()

                x_tile_copy(t, slot).wait()
                xt = xv_ref[slot]
                for half in range(2):
                    y = jnp.dot(
                        xt, wv_ref[half], preferred_element_type=jnp.float32
                    )
                    yv_ref[half, pl.ds(t * TM, TM), :] = y * jax.nn.sigmoid(y)
                return 0

            lax.fori_loop(0, n_xt, tile_body, 0)

        hop = start_hop(0)
        window(my, my)
        out_copies = []
        for half in range(2):
            c = pltpu.make_async_copy(
                yv_ref.at[half],
                out_ref.at[pl.ds(my * m_x, m_x), pl.ds(half * nh, nh)],
                out_sems.at[half],
            )
            c.start()
            out_copies.append(c)
        hop[0].wait()
        hop[1].wait()

        hop = start_hop(1)
        for c in out_copies:
            c.wait()
        window(left, right)
        send_a = a2a_descriptor(0, left, my, 0)
        send_b = a2a_descriptor(5, right, my, 1)
        send_a.start()
        send_b.start()
        hop[0].wait()
        hop[1].wait()

        hop = start_hop(2)
        send_a.wait_send()
        send_b.wait_send()
        window(opp, opp)
        send_a = a2a_descriptor(2, opp, my, 0)
        send_b = a2a_descriptor(3, opp, my, 1)
        send_a.start()
        send_b.start()
        hop[0].wait()
        hop[1].wait()

        send_a.wait_send()
        send_b.wait_send()
        window(right, left)
        send_a = a2a_descriptor(4, right, my, 0)
        send_b = a2a_descriptor(1, left, my, 1)
        send_a.start()
        send_b.start()
        send_a.wait_send()
        send_b.wait_send()

        for rel in range(1, N_DEV):
            s = lax.rem(my + rel, N_DEV)
            for half in range(2):
                slot = (rel - 1) * 2 + half
                recv = pltpu.make_async_remote_copy(
                    src_ref=yv_ref.at[half],
                    dst_ref=out_ref.at[pl.ds(s * m_x, m_x),
                                       pl.ds(half * nh, nh)],
                    send_sem=a2a_send.at[slot],
                    recv_sem=a2a_recv.at[slot],
                    device_id=(my,),
                    device_id_type=pl.DeviceIdType.MESH,
                )
                recv.wait_recv()

    out, _ = pl.pallas_call(
        body,
        out_shape=(
            jax.ShapeDtypeStruct((N_DEV * m_x, n_w), jnp.float32),
            jax.ShapeDtypeStruct((N_DEV, 2, k, nh), jnp.float32),
        ),
        in_specs=[
            pl.BlockSpec(memory_space=pl.ANY),
            pl.BlockSpec(memory_space=pl.ANY),
        ],
        out_specs=(
            pl.BlockSpec(memory_space=pl.ANY),
            pl.BlockSpec(memory_space=pl.ANY),
        ),
        scratch_shapes=[
            pltpu.MemorySpace.VMEM((2, k, nh), jnp.float32),
            pltpu.MemorySpace.VMEM((2, TM, k), jnp.float32),
            pltpu.MemorySpace.VMEM((2, m_x, nh), jnp.float32),
            pltpu.SemaphoreType.DMA((2,)),
            pltpu.SemaphoreType.DMA((2,)),
            pltpu.SemaphoreType.DMA((2,)),
            pltpu.SemaphoreType.DMA((2,)),
            pltpu.SemaphoreType.DMA((N_DEV - 1,)),
            pltpu.SemaphoreType.DMA((N_DEV - 1,)),
            pltpu.SemaphoreType.DMA((N_DEV - 1,)),
            pltpu.SemaphoreType.DMA((N_DEV - 1,)),
            pltpu.SemaphoreType.DMA((6,)),
            pltpu.SemaphoreType.DMA((6,)),
        ],
        compiler_params=pltpu.CompilerParams(
            collective_id=0,
            vmem_limit_bytes=100 * 1024 * 1024,
        ),
    )(x, w_mat)
    return out


# device time: 1783985 ns/iter; 2.4521x vs baseline; 1.0210x over previous
import jax
import jax.numpy as jnp
from jax import lax
from jax.experimental import pallas as pl
from jax.experimental.pallas import tpu as pltpu

N_DEV = 4
TM = 128


def kernel(x, w_mat):
    m_x, k = x.shape
    _, n_w = w_mat.shape
    nh = n_w // 2
    n_xt = m_x // TM

    def body(x_ref, w_ref, out_ref, wg_ref, rly_ref, wv_ref, xv_ref, yv_ref,
             split_sems, wv_sems, xv_sems, out_sems,
             send_r, recv_r, send_l, recv_l, a2a_send, a2a_recv,
             rly_send, rly_recv):
        my = lax.axis_index("i")
        left = lax.rem(my + N_DEV - 1, N_DEV)
        right = lax.rem(my + 1, N_DEV)
        opp = lax.rem(my + 2, N_DEV)

        barrier_sem = pltpu.get_barrier_semaphore()
        for nbr in (left, right):
            pl.semaphore_signal(
                barrier_sem, inc=1,
                device_id=(nbr,), device_id_type=pl.DeviceIdType.MESH,
            )
        pl.semaphore_wait(barrier_sem, 2)

        splits = []
        for half in range(2):
            c = pltpu.make_async_copy(
                w_ref.at[:, pl.ds(half * nh, nh)],
                wg_ref.at[my, half],
                split_sems.at[half],
            )
            c.start()
            splits.append(c)
        for c in splits:
            c.wait()

        def start_hop(h):
            o_r = lax.rem(my + N_DEV - h, N_DEV)
            o_l = lax.rem(my + h, N_DEV)
            rdma_r = pltpu.make_async_remote_copy(
                src_ref=wg_ref.at[o_r, 0],
                dst_ref=wg_ref.at[o_r, 0],
                send_sem=send_r.at[h],
                recv_sem=recv_r.at[h],
                device_id=(right,),
                device_id_type=pl.DeviceIdType.MESH,
            )
            rdma_l = pltpu.make_async_remote_copy(
                src_ref=wg_ref.at[o_l, 1],
                dst_ref=wg_ref.at[o_l, 1],
                send_sem=send_l.at[h],
                recv_sem=recv_l.at[h],
                device_id=(left,),
                device_id_type=pl.DeviceIdType.MESH,
            )
            rdma_r.start()
            rdma_l.start()
            return rdma_r, rdma_l

        def a2a_descriptor(slot, target, src_rows, half):
            return pltpu.make_async_remote_copy(
                src_ref=yv_ref.at[half],
                dst_ref=out_ref.at[pl.ds(src_rows * m_x, m_x),
                                   pl.ds(half * nh, nh)],
                send_sem=a2a_send.at[slot],
                recv_sem=a2a_recv.at[slot],
                device_id=(target,),
                device_id_type=pl.DeviceIdType.MESH,
            )

        def window(i_a, i_b):
            cw = []
            for half, i_dev in ((0, i_a), (1, i_b)):
                c = pltpu.make_async_copy(
                    wg_ref.at[i_dev, half], wv_ref.at[half], wv_sems.at[half]
                )
                c.start()
                cw.append(c)
            def x_tile_copy(t, slot):
                return pltpu.make_async_copy(
                    x_ref.at[pl.ds(t * TM, TM), :],
                    xv_ref.at[slot],
                    xv_sems.at[slot],
                )

            x_tile_copy(0, 0).start()
            for c in cw:
                c.wait()

            def tile_body(t, _):
                slot = lax.rem(t, 2)
                nxt_slot = lax.rem(t + 1, 2)

                @pl.when(t + 1 < n_xt)
                def _():
                    x_tile_copy(t + 1, nxt_slot).start()

                x_tile_copy(t, slot).wait()
                xt = xv_ref[slot]
                for half in range(2):
                    y = jnp.dot(
                        xt, wv_ref[half], preferred_element_type=jnp.float32
                    )
                    yv_ref[half, pl.ds(t * TM, TM), :] = y * jax.nn.sigmoid(y)
                return 0

            lax.fori_loop(0, n_xt, tile_body, 0)

        hop = start_hop(0)
        window(my, my)
        out_copies = []
        for half in range(2):
            c = pltpu.make_async_copy(
                yv_ref.at[half],
                out_ref.at[pl.ds(my * m_x, m_x), pl.ds(half * nh, nh)],
                out_sems.at[half],
            )
            c.start()
            out_copies.append(c)
        hop[0].wait()
        hop[1].wait()

        hop = start_hop(1)
        for c in out_copies:
            c.wait()
        window(left, right)
        send_a = a2a_descriptor(0, left, my, 0)
        send_b = a2a_descriptor(5, right, my, 1)
        send_a.start()
        send_b.start()
        hop[0].wait()
        hop[1].wait()

        hop = start_hop(2)
        send_a.wait_send()
        send_b.wait_send()
        window(opp, opp)
        send_a = pltpu.make_async_remote_copy(
            src_ref=yv_ref.at[0],
            dst_ref=rly_ref.at[0],
            send_sem=rly_send.at[0],
            recv_sem=rly_recv.at[0],
            device_id=(right,),
            device_id_type=pl.DeviceIdType.MESH,
        )
        send_b = pltpu.make_async_remote_copy(
            src_ref=yv_ref.at[1],
            dst_ref=rly_ref.at[1],
            send_sem=rly_send.at[1],
            recv_sem=rly_recv.at[1],
            device_id=(left,),
            device_id_type=pl.DeviceIdType.MESH,
        )
        send_a.start()
        send_b.start()
        hop[0].wait()
        hop[1].wait()

        send_a.wait_recv()
        send_b.wait_recv()
        fwd_a = pltpu.make_async_remote_copy(
            src_ref=rly_ref.at[0],
            dst_ref=out_ref.at[pl.ds(left * m_x, m_x), pl.ds(0, nh)],
            send_sem=a2a_send.at[2],
            recv_sem=a2a_recv.at[2],
            device_id=(right,),
            device_id_type=pl.DeviceIdType.MESH,
        )
        fwd_b = pltpu.make_async_remote_copy(
            src_ref=rly_ref.at[1],
            dst_ref=out_ref.at[pl.ds(right * m_x, m_x), pl.ds(nh, nh)],
            send_sem=a2a_send.at[3],
            recv_sem=a2a_recv.at[3],
            device_id=(left,),
            device_id_type=pl.DeviceIdType.MESH,
        )
        fwd_a.start()
        fwd_b.start()

        send_a.wait_send()
        send_b.wait_send()
        window(right, left)
        send_a = a2a_descriptor(4, right, my, 0)
        send_b = a2a_descriptor(1, left, my, 1)
        send_a.start()
        send_b.start()
        send_a.wait_send()
        send_b.wait_send()
        fwd_a.wait_send()
        fwd_b.wait_send()

        for rel in range(1, N_DEV):
            s = lax.rem(my + rel, N_DEV)
            for half in range(2):
                slot = (rel - 1) * 2 + half
                recv = pltpu.make_async_remote_copy(
                    src_ref=yv_ref.at[half],
                    dst_ref=out_ref.at[pl.ds(s * m_x, m_x),
                                       pl.ds(half * nh, nh)],
                    send_sem=a2a_send.at[slot],
                    recv_sem=a2a_recv.at[slot],
                    device_id=(my,),
                    device_id_type=pl.DeviceIdType.MESH,
                )
                recv.wait_recv()

    out, _, _ = pl.pallas_call(
        body,
        out_shape=(
            jax.ShapeDtypeStruct((N_DEV * m_x, n_w), jnp.float32),
            jax.ShapeDtypeStruct((N_DEV, 2, k, nh), jnp.float32),
            jax.ShapeDtypeStruct((2, m_x, nh), jnp.float32),
        ),
        in_specs=[
            pl.BlockSpec(memory_space=pl.ANY),
            pl.BlockSpec(memory_space=pl.ANY),
        ],
        out_specs=(
            pl.BlockSpec(memory_space=pl.ANY),
            pl.BlockSpec(memory_space=pl.ANY),
            pl.BlockSpec(memory_space=pl.ANY),
        ),
        scratch_shapes=[
            pltpu.MemorySpace.VMEM((2, k, nh), jnp.float32),
            pltpu.MemorySpace.VMEM((2, TM, k), jnp.float32),
            pltpu.MemorySpace.VMEM((2, m_x, nh), jnp.float32),
            pltpu.SemaphoreType.DMA((2,)),
            pltpu.SemaphoreType.DMA((2,)),
            pltpu.SemaphoreType.DMA((2,)),
            pltpu.SemaphoreType.DMA((2,)),
            pltpu.SemaphoreType.DMA((N_DEV - 1,)),
            pltpu.SemaphoreType.DMA((N_DEV - 1,)),
            pltpu.SemaphoreType.DMA((N_DEV - 1,)),
            pltpu.SemaphoreType.DMA((N_DEV - 1,)),
            pltpu.SemaphoreType.DMA((6,)),
            pltpu.SemaphoreType.DMA((6,)),
            pltpu.SemaphoreType.DMA((2,)),
            pltpu.SemaphoreType.DMA((2,)),
        ],
        compiler_params=pltpu.CompilerParams(
            collective_id=0,
            vmem_limit_bytes=100 * 1024 * 1024,
        ),
    )(x, w_mat)
    return out


# device time: 783547 ns/iter; 5.5831x vs baseline; 2.2768x over previous
import jax
import jax.numpy as jnp
from jax import lax
from jax.experimental import pallas as pl
from jax.experimental.pallas import tpu as pltpu

N_DEV = 4
TM = 128


def kernel(x, w_mat):
    m_x, k = x.shape
    _, n_w = w_mat.shape
    nh = n_w // 2
    n_xt = m_x // TM

    def body(x_ref, w_ref, out_ref, wg_ref, rly_ref, wv_ref, xv_ref, yv_ref,
             wv_sems, xv_sems, out_sems,
             send_r, recv_r, send_l, recv_l, a2a_send, a2a_recv,
             rly_send, rly_recv):
        my = lax.axis_index("i")
        left = lax.rem(my + N_DEV - 1, N_DEV)
        right = lax.rem(my + 1, N_DEV)
        opp = lax.rem(my + 2, N_DEV)

        barrier_sem = pltpu.get_barrier_semaphore()
        for nbr in (left, right):
            pl.semaphore_signal(
                barrier_sem, inc=1,
                device_id=(nbr,), device_id_type=pl.DeviceIdType.MESH,
            )
        pl.semaphore_wait(barrier_sem, 2)

        def start_hop(h):
            o_r = lax.rem(my + N_DEV - h, N_DEV)
            o_l = lax.rem(my + h, N_DEV)
            src_r = w_ref.at[:, pl.ds(0, nh)] if h == 0 else wg_ref.at[o_r, 0]
            src_l = w_ref.at[:, pl.ds(nh, nh)] if h == 0 else wg_ref.at[o_l, 1]
            rdma_r = pltpu.make_async_remote_copy(
                src_ref=src_r,
                dst_ref=wg_ref.at[o_r, 0],
                send_sem=send_r.at[h],
                recv_sem=recv_r.at[h],
                device_id=(right,),
                device_id_type=pl.DeviceIdType.MESH,
            )
            rdma_l = pltpu.make_async_remote_copy(
                src_ref=src_l,
                dst_ref=wg_ref.at[o_l, 1],
                send_sem=send_l.at[h],
                recv_sem=recv_l.at[h],
                device_id=(left,),
                device_id_type=pl.DeviceIdType.MESH,
            )
            rdma_r.start()
            rdma_l.start()
            return rdma_r, rdma_l

        def a2a_descriptor(slot, target, src_rows, half):
            return pltpu.make_async_remote_copy(
                src_ref=yv_ref.at[half],
                dst_ref=out_ref.at[pl.ds(src_rows * m_x, m_x),
                                   pl.ds(half * nh, nh)],
                send_sem=a2a_send.at[slot],
                recv_sem=a2a_recv.at[slot],
                device_id=(target,),
                device_id_type=pl.DeviceIdType.MESH,
            )

        def window(i_a, i_b, local=False):
            cw = []
            for half, i_dev in ((0, i_a), (1, i_b)):
                wsrc = (w_ref.at[:, pl.ds(half * nh, nh)] if local
                        else wg_ref.at[i_dev, half])
                c = pltpu.make_async_copy(
                    wsrc, wv_ref.at[half], wv_sems.at[half]
                )
                c.start()
                cw.append(c)
            def x_tile_copy(t, slot):
                return pltpu.make_async_copy(
                    x_ref.at[pl.ds(t * TM, TM), :],
                    xv_ref.at[slot],
                    xv_sems.at[slot],
                )

            x_tile_copy(0, 0).start()
            for c in cw:
                c.wait()

            def tile_body(t, _):
                slot = lax.rem(t, 2)
                nxt_slot = lax.rem(t + 1, 2)

                @pl.when(t + 1 < n_xt)
                def _():
                    x_tile_copy(t + 1, nxt_slot).start()

                x_tile_copy(t, slot).wait()
                xt = xv_ref[slot]
                for half in range(2):
                    y = jnp.dot(
                        xt, wv_ref[half], preferred_element_type=jnp.float32
                    )
                    yv_ref[half, pl.ds(t * TM, TM), :] = y * jax.nn.sigmoid(y)
                return 0

            lax.fori_loop(0, n_xt, tile_body, 0)

        def window_half(i_dev, half):
            cw = pltpu.make_async_copy(
                wg_ref.at[i_dev, half], wv_ref.at[half], wv_sems.at[half]
            )
            cw.start()

            def x_tile_copy(t, slot):
                return pltpu.make_async_copy(
                    x_ref.at[pl.ds(t * TM, TM), :],
                    xv_ref.at[slot],
                    xv_sems.at[slot],
                )

            x_tile_copy(0, 0).start()
            cw.wait()

            def tile_body(t, _):
                slot = lax.rem(t, 2)
                nxt_slot = lax.rem(t + 1, 2)

                @pl.when(t + 1 < n_xt)
                def _():
                    x_tile_copy(t + 1, nxt_slot).start()

                x_tile_copy(t, slot).wait()
                y = jnp.dot(
                    xv_ref[slot], wv_ref[half],
                    preferred_element_type=jnp.float32,
                )
                yv_ref[half, pl.ds(t * TM, TM), :] = y * jax.nn.sigmoid(y)
                return 0

            lax.fori_loop(0, n_xt, tile_body, 0)

        hop = start_hop(0)
        window(my, my, local=True)
        out_copies = []
        for half in range(2):
            c = pltpu.make_async_copy(
                yv_ref.at[half],
                out_ref.at[pl.ds(my * m_x, m_x), pl.ds(half * nh, nh)],
                out_sems.at[half],
            )
            c.start()
            out_copies.append(c)
        hop[0].wait()
        hop[1].wait()

        hop = start_hop(1)
        for c in out_copies:
            c.wait()
        window(left, right)
        send_a = a2a_descriptor(0, left, my, 0)
        send_b = a2a_descriptor(5, right, my, 1)
        send_a.start()
        send_b.start()
        hop[0].wait()
        hop[1].wait()

        hop = start_hop(2)
        send_a.wait_send()
        send_b.wait_send()
        window(opp, opp)
        send_a = pltpu.make_async_remote_copy(
            src_ref=yv_ref.at[0],
            dst_ref=rly_ref.at[0],
            send_sem=rly_send.at[0],
            recv_sem=rly_recv.at[0],
            device_id=(right,),
            device_id_type=pl.DeviceIdType.MESH,
        )
        send_b = pltpu.make_async_remote_copy(
            src_ref=yv_ref.at[1],
            dst_ref=rly_ref.at[1],
            send_sem=rly_send.at[1],
            recv_sem=rly_recv.at[1],
            device_id=(left,),
            device_id_type=pl.DeviceIdType.MESH,
        )
        send_a.start()
        send_b.start()
        hop[0].wait()
        hop[1].wait()

        send_a.wait_recv()
        send_b.wait_recv()
        fwd_a = pltpu.make_async_remote_copy(
            src_ref=rly_ref.at[0],
            dst_ref=out_ref.at[pl.ds(left * m_x, m_x), pl.ds(0, nh)],
            send_sem=a2a_send.at[2],
            recv_sem=a2a_recv.at[2],
            device_id=(right,),
            device_id_type=pl.DeviceIdType.MESH,
        )
        fwd_b = pltpu.make_async_remote_copy(
            src_ref=rly_ref.at[1],
            dst_ref=out_ref.at[pl.ds(right * m_x, m_x), pl.ds(nh, nh)],
            send_sem=a2a_send.at[3],
            recv_sem=a2a_recv.at[3],
            device_id=(left,),
            device_id_type=pl.DeviceIdType.MESH,
        )
        fwd_a.start()
        fwd_b.start()

        send_a.wait_send()
        send_b.wait_send()
        window_half(right, 0)
        send_a = a2a_descriptor(4, right, my, 0)
        send_a.start()
        window_half(left, 1)
        send_b = a2a_descriptor(1, left, my, 1)
        send_b.start()
        send_a.wait_send()
        send_b.wait_send()
        fwd_a.wait_send()
        fwd_b.wait_send()

        for rel in range(1, N_DEV):
            s = lax.rem(my + rel, N_DEV)
            for half in range(2):
                slot = (rel - 1) * 2 + half
                recv = pltpu.make_async_remote_copy(
                    src_ref=yv_ref.at[half],
                    dst_ref=out_ref.at[pl.ds(s * m_x, m_x),
                                       pl.ds(half * nh, nh)],
                    send_sem=a2a_send.at[slot],
                    recv_sem=a2a_recv.at[slot],
                    device_id=(my,),
                    device_id_type=pl.DeviceIdType.MESH,
                )
                recv.wait_recv()

    out, _, _ = pl.pallas_call(
        body,
        out_shape=(
            jax.ShapeDtypeStruct((N_DEV * m_x, n_w), jnp.float32),
            jax.ShapeDtypeStruct((N_DEV, 2, k, nh), jnp.float32),
            jax.ShapeDtypeStruct((2, m_x, nh), jnp.float32),
        ),
        in_specs=[
            pl.BlockSpec(memory_space=pl.ANY),
            pl.BlockSpec(memory_space=pl.ANY),
        ],
        out_specs=(
            pl.BlockSpec(memory_space=pl.ANY),
            pl.BlockSpec(memory_space=pl.ANY),
            pl.BlockSpec(memory_space=pl.ANY),
        ),
        scratch_shapes=[
            pltpu.MemorySpace.VMEM((2, k, nh), jnp.float32),
            pltpu.MemorySpace.VMEM((2, TM, k), jnp.float32),
            pltpu.MemorySpace.VMEM((2, m_x, nh), jnp.float32),
            pltpu.SemaphoreType.DMA((2,)),
            pltpu.SemaphoreType.DMA((2,)),
            pltpu.SemaphoreType.DMA((2,)),
            pltpu.SemaphoreType.DMA((N_DEV - 1,)),
            pltpu.SemaphoreType.DMA((N_DEV - 1,)),
            pltpu.SemaphoreType.DMA((N_DEV - 1,)),
            pltpu.SemaphoreType.DMA((N_DEV - 1,)),
            pltpu.SemaphoreType.DMA((6,)),
            pltpu.SemaphoreType.DMA((6,)),
            pltpu.SemaphoreType.DMA((2,)),
            pltpu.SemaphoreType.DMA((2,)),
        ],
        compiler_params=pltpu.CompilerParams(
            collective_id=0,
            vmem_limit_bytes=100 * 1024 * 1024,
        ),
    )(x, w_mat)
    return out
